# baseline (speedup 1.0000x reference)
import jax
import jax.numpy as jnp
from jax import lax
from jax.experimental import pallas as pl
from jax.experimental.pallas import tpu as pltpu
from jax.experimental.pallas import tpu_sc as plsc

N_NODES = 10000
N_EDGES = 320000
N_REL = 2
DIM = 128
NUM_TILES = 16
EDGES_PER_TILE = N_EDGES // NUM_TILES
CHUNK = 80
HALF = CHUNK // 2
N_CHUNKS = EDGES_PER_TILE // CHUNK
NBUF = 4
ROWS_PER_TILE = 624
TAIL_ROW = ROWS_PER_TILE * NUM_TILES
TAIL = N_NODES - TAIL_ROW


def _sc_segment_body(vertex, edata, out,
                     e0, e1, e2, e3,
                     r0, r1, r2, r3, acc,
                     gA, gB, gC, gD, sA, sB, sC, sD, iA, iB):
    r = lax.axis_index("c")
    t = lax.axis_index("s")
    eb = (e0, e1, e2, e3)
    rows = (r0, r1, r2, r3)
    gsem = (gA, gB, gC, gD)
    ssem = (sA, sB, sC, sD)
    isem = (iA, iB)

    def zero_row(i, _):
        for f in range(DIM // 16):
            r0[i, pl.ds(f * 16, 16)] = jnp.zeros((16,), jnp.float32)
        return 0
    lax.fori_loop(0, CHUNK, zero_row, 0)
    row_base = t * ROWS_PER_TILE
    for k in range(7):
        pltpu.sync_copy(r0, acc.at[pl.ds(row_base + k * CHUNK, CHUNK)])
    pltpu.sync_copy(r0.at[pl.ds(0, 64)], acc.at[pl.ds(row_base + 560, 64)])

    @pl.when(t == 0)
    def _zero_tail():
        pltpu.sync_copy(r0.at[pl.ds(0, TAIL)], acc.at[pl.ds(TAIL_ROW, TAIL)])

    plsc.subcore_barrier()

    def issue_idx(c, i2, b4):
        pltpu.async_copy(edata.at[r, t, c], eb[b4], isem[i2])

    def wait_idx(i2):
        pltpu.make_async_copy(edata.at[0, 0, 0], eb[0], isem[i2]).wait()

    def issue_gather(b4):
        pltpu.async_copy(vertex.at[eb[b4].at[0]], rows[b4], gsem[b4])

    def wait_gather(b4):
        pltpu.make_async_copy(vertex.at[pl.ds(0, CHUNK)], rows[b4],
                              gsem[b4]).wait()

    def issue_scatter(b4):
        pltpu.async_copy(rows[b4], acc.at[eb[b4].at[1]], ssem[b4], add=True)

    def wait_scatter(b4):
        pltpu.make_async_copy(rows[b4], acc.at[pl.ds(0, CHUNK)],
                              ssem[b4]).wait()

    def scale(b4):
        def grp(g2, _):
            w_vec = lax.bitcast_convert_type(
                eb[b4][2, pl.ds(g2 * 16, 16)], jnp.float32)
            for j in range(16):
                wj = w_vec[j]
                e = g2 * 16 + j
                for f in range(DIM // 16):
                    sl = pl.ds(f * 16, 16)
                    rows[b4][e, sl] = rows[b4][e, sl] * wj
            return 0
        lax.fori_loop(0, CHUNK // 16, grp, 0)

    issue_idx(0, 0, 0)
    issue_idx(1, 1, 1)
    wait_idx(0)
    issue_gather(0)

    def outer(g, _):
        for b in range(NBUF):
            c = NBUF * g + b
            i2 = b % 2
            inx = (b + 1) % 2

            @pl.when(c >= 2)
            def _wait_prev_scatter():
                wait_scatter((b + 2) % NBUF)

            @pl.when(c + 1 < N_CHUNKS)
            def _launch_next_gather():
                wait_idx(inx)
                issue_gather((b + 1) % NBUF)

            @pl.when(c < N_CHUNKS)
            def _scale_and_scatter():
                wait_gather(b)
                scale(b)
                issue_scatter(b)

            @pl.when(c + 2 < N_CHUNKS)
            def _prefetch_idx():
                issue_idx(c + 2, i2, (b + 2) % NBUF)
        return 0

    lax.fori_loop(0, N_CHUNKS // NBUF + 1, outer, 0)
    plsc.subcore_barrier()

    for k in range(7):
        row0 = row_base + k * CHUNK
        pltpu.sync_copy(acc.at[pl.ds(row0, CHUNK)], r0)
        pltpu.sync_copy(r0, out.at[r, pl.ds(row0, CHUNK)])
    pltpu.sync_copy(acc.at[pl.ds(row_base + 560, 64)], r0.at[pl.ds(0, 64)])
    pltpu.sync_copy(r0.at[pl.ds(0, 64)], out.at[r, pl.ds(row_base + 560, 64)])

    @pl.when(t == 0)
    def _flush_tail():
        pltpu.sync_copy(acc.at[pl.ds(TAIL_ROW, TAIL)], r1.at[pl.ds(0, TAIL)])
        pltpu.sync_copy(r1.at[pl.ds(0, TAIL)], out.at[r, pl.ds(TAIL_ROW, TAIL)])


@jax.jit
def _sc_segment(vertex, edata):
    mesh = plsc.VectorSubcoreMesh(core_axis_name="c", subcore_axis_name="s")
    return pl.kernel(
        _sc_segment_body,
        out_type=jax.ShapeDtypeStruct((N_REL, N_NODES, DIM), jnp.float32),
        mesh=mesh,
        scratch_types=[
            pltpu.VMEM((3, CHUNK), jnp.int32),
            pltpu.VMEM((3, CHUNK), jnp.int32),
            pltpu.VMEM((3, CHUNK), jnp.int32),
            pltpu.VMEM((3, CHUNK), jnp.int32),
            pltpu.VMEM((CHUNK, DIM), jnp.float32),
            pltpu.VMEM((CHUNK, DIM), jnp.float32),
            pltpu.VMEM((CHUNK, DIM), jnp.float32),
            pltpu.VMEM((CHUNK, DIM), jnp.float32),
            pltpu.VMEM_SHARED((N_NODES, DIM), jnp.float32),
            pltpu.SemaphoreType.DMA,
            pltpu.SemaphoreType.DMA,
            pltpu.SemaphoreType.DMA,
            pltpu.SemaphoreType.DMA,
            pltpu.SemaphoreType.DMA,
            pltpu.SemaphoreType.DMA,
            pltpu.SemaphoreType.DMA,
            pltpu.SemaphoreType.DMA,
            pltpu.SemaphoreType.DMA,
            pltpu.SemaphoreType.DMA,
        ],
    )(vertex, edata)


def _tc_matmul_body(sup_ref, w_ref, wc_ref, b_ref, o_ref):
    c00 = wc_ref[0, 0]
    c01 = wc_ref[0, 1]
    c10 = wc_ref[1, 0]
    c11 = wc_ref[1, 1]
    a0 = w_ref[0:DIM, :]
    a1 = w_ref[DIM:2 * DIM, :]
    p0 = c00 * a0 + c01 * a1
    p1 = c10 * a0 + c11 * a1
    pp = jnp.concatenate([p0, p1], axis=0)
    rows_i = lax.broadcasted_iota(jnp.int32, (2 * DIM, 2 * DIM), 0)
    cols_i = lax.broadcasted_iota(jnp.int32, (2 * DIM, 2 * DIM), 1)
    g = ((rows_i % 2) * DIM + rows_i // 2 == cols_i).astype(jnp.float32)
    veff = jnp.dot(g, pp, preferred_element_type=jnp.float32)
    u0 = veff[0:DIM, :]
    u1 = veff[DIM:2 * DIM, :]
    acc = jnp.dot(sup_ref[0], u0, preferred_element_type=jnp.float32)
    acc = acc + jnp.dot(sup_ref[1], u1, preferred_element_type=jnp.float32)
    o_ref[...] = acc + b_ref[...]


@jax.jit
def _tc_matmul(supports, W, W_comp, B2d):
    blk = 2000
    return pl.pallas_call(
        _tc_matmul_body,
        grid=(N_NODES // blk,),
        in_specs=[
            pl.BlockSpec((N_REL, blk, DIM), lambda i: (0, i, 0)),
            pl.BlockSpec((2 * DIM, DIM), lambda i: (0, 0)),
            pl.BlockSpec(memory_space=pltpu.SMEM),
            pl.BlockSpec((1, DIM), lambda i: (0, 0)),
        ],
        out_specs=pl.BlockSpec((blk, DIM), lambda i: (i, 0)),
        out_shape=jax.ShapeDtypeStruct((N_NODES, DIM), jnp.float32),
    )(supports, W, W_comp, B2d)


def kernel(vertex, edge_index, edge_weight, W, W_comp, B):
    ei = edge_index.astype(jnp.int32)
    shp = (N_REL, NUM_TILES, N_CHUNKS, CHUNK)
    src4 = ei[:, 0, :].reshape(shp)
    dst4 = ei[:, 1, :].reshape(shp)
    w4 = lax.bitcast_convert_type(edge_weight, jnp.int32).reshape(shp)
    edata = jnp.stack([src4, dst4, w4], axis=3)
    supports = _sc_segment(vertex, edata)
    return _tc_matmul(supports, W, W_comp, B.reshape(1, DIM))


# speedup vs baseline: 1.1119x; 1.1119x over previous
import jax
import jax.numpy as jnp
from jax import lax
from jax.experimental import pallas as pl
from jax.experimental.pallas import tpu as pltpu
from jax.experimental.pallas import tpu_sc as plsc

N_NODES = 10000
N_EDGES = 320000
N_REL = 2
DIM = 128
NUM_TILES = 16
EDGES_PER_TILE = N_EDGES // NUM_TILES
CHUNK = 80
HALF_A = 48
HALF_B = CHUNK - HALF_A
N_CHUNKS = EDGES_PER_TILE // CHUNK
NBUF = 4
ROWS_PER_TILE = 624
TAIL_ROW = ROWS_PER_TILE * NUM_TILES
TAIL = N_NODES - TAIL_ROW


def _sc_segment_body(vertex, src, dst, ew, out,
                     s0a, s0b, s1a, s1b, d0, d1, d2, d3, w0, w1,
                     r0, r1, r2, r3, acc,
                     gA0, gA1, gA2, gA3, gB0, gB1, gB2, gB3,
                     sA, sB, sC, sD, iA, iB):
    r = lax.axis_index("c")
    t = lax.axis_index("s")
    srcb = ((s0a, s0b), (s1a, s1b))
    dstb = (d0, d1, d2, d3)
    wb = (w0, w1)
    rows = (r0, r1, r2, r3)
    gsemA = (gA0, gA1, gA2, gA3)
    gsemB = (gB0, gB1, gB2, gB3)
    ssem = (sA, sB, sC, sD)
    isem = (iA, iB)
    base = r * N_EDGES + t * EDGES_PER_TILE

    def zero_row(i, _):
        for f in range(DIM // 16):
            r0[i, pl.ds(f * 16, 16)] = jnp.zeros((16,), jnp.float32)
        return 0
    lax.fori_loop(0, CHUNK, zero_row, 0)
    row_base = t * ROWS_PER_TILE
    for k in range(7):
        pltpu.sync_copy(r0, acc.at[pl.ds(row_base + k * CHUNK, CHUNK)])
    pltpu.sync_copy(r0.at[pl.ds(0, 64)], acc.at[pl.ds(row_base + 560, 64)])

    @pl.when(t == 0)
    def _zero_tail():
        pltpu.sync_copy(r0.at[pl.ds(0, TAIL)], acc.at[pl.ds(TAIL_ROW, TAIL)])

    plsc.subcore_barrier()

    def issue_idx(c, i2, b4):
        off = base + c * CHUNK
        pltpu.async_copy(src.at[pl.ds(off, HALF_A)], srcb[i2][0], isem[i2])
        pltpu.async_copy(src.at[pl.ds(off + HALF_A, HALF_B)], srcb[i2][1],
                         isem[i2])
        pltpu.async_copy(dst.at[pl.ds(off, CHUNK)], dstb[b4], isem[i2])
        pltpu.async_copy(ew.at[pl.ds(off, CHUNK)], wb[i2], isem[i2])

    def wait_idx(i2):
        pltpu.make_async_copy(src.at[pl.ds(0, HALF_A)], srcb[i2][0],
                              isem[i2]).wait()
        pltpu.make_async_copy(src.at[pl.ds(0, HALF_B)], srcb[i2][1],
                              isem[i2]).wait()
        pltpu.make_async_copy(dst.at[pl.ds(0, CHUNK)], dstb[0], isem[i2]).wait()
        pltpu.make_async_copy(ew.at[pl.ds(0, CHUNK)], wb[i2], isem[i2]).wait()

    def issue_gather(i2, b4):
        pltpu.async_copy(vertex.at[srcb[i2][0]],
                         rows[b4].at[pl.ds(0, HALF_A)], gsemA[b4])
        pltpu.async_copy(vertex.at[srcb[i2][1]],
                         rows[b4].at[pl.ds(HALF_A, HALF_B)], gsemB[b4])

    def wait_gather_a(b4):
        pltpu.make_async_copy(vertex.at[pl.ds(0, HALF_A)],
                              rows[b4].at[pl.ds(0, HALF_A)], gsemA[b4]).wait()

    def wait_gather_b(b4):
        pltpu.make_async_copy(vertex.at[pl.ds(0, HALF_B)],
                              rows[b4].at[pl.ds(0, HALF_B)], gsemB[b4]).wait()

    def issue_scatter(b4):
        pltpu.async_copy(rows[b4], acc.at[dstb[b4]], ssem[b4], add=True)

    def wait_scatter(b4):
        pltpu.make_async_copy(rows[b4], acc.at[pl.ds(0, CHUNK)],
                              ssem[b4]).wait()

    def scale(i2, b4, g_lo, g_hi):
        def grp(g2, _):
            w_vec = wb[i2][pl.ds(g2 * 16, 16)]
            for j in range(16):
                wj = w_vec[j]
                e = g2 * 16 + j
                for f in range(DIM // 16):
                    sl = pl.ds(f * 16, 16)
                    rows[b4][e, sl] = rows[b4][e, sl] * wj
            return 0
        lax.fori_loop(g_lo, g_hi, grp, 0)

    issue_idx(0, 0, 0)
    issue_idx(1, 1, 1)
    wait_idx(0)
    issue_gather(0, 0)

    def outer(g, _):
        for b in range(NBUF):
            c = NBUF * g + b
            i2 = b % 2
            inx = (b + 1) % 2

            @pl.when(c >= 2)
            def _wait_prev_scatter():
                wait_scatter((b + 2) % NBUF)

            @pl.when(c + 1 < N_CHUNKS)
            def _launch_next_gather():
                wait_idx(inx)
                issue_gather(inx, (b + 1) % NBUF)

            @pl.when(c < N_CHUNKS)
            def _scale_and_scatter():
                wait_gather_a(b)
                scale(i2, b, 0, HALF_A // 16)
                wait_gather_b(b)
                scale(i2, b, HALF_A // 16, CHUNK // 16)
                issue_scatter(b)

            @pl.when(c + 2 < N_CHUNKS)
            def _prefetch_idx():
                issue_idx(c + 2, i2, (b + 2) % NBUF)
        return 0

    lax.fori_loop(0, N_CHUNKS // NBUF + 1, outer, 0)
    plsc.subcore_barrier()

    for k in range(7):
        row0 = row_base + k * CHUNK
        pltpu.sync_copy(acc.at[pl.ds(row0, CHUNK)], r0)
        pltpu.sync_copy(r0, out.at[r, pl.ds(row0, CHUNK)])
    pltpu.sync_copy(acc.at[pl.ds(row_base + 560, 64)], r0.at[pl.ds(0, 64)])
    pltpu.sync_copy(r0.at[pl.ds(0, 64)], out.at[r, pl.ds(row_base + 560, 64)])

    @pl.when(t == 0)
    def _flush_tail():
        pltpu.sync_copy(acc.at[pl.ds(TAIL_ROW, TAIL)], r1.at[pl.ds(0, TAIL)])
        pltpu.sync_copy(r1.at[pl.ds(0, TAIL)], out.at[r, pl.ds(TAIL_ROW, TAIL)])


@jax.jit
def _sc_segment(vertex, src, dst, ew):
    mesh = plsc.VectorSubcoreMesh(core_axis_name="c", subcore_axis_name="s")
    return pl.kernel(
        _sc_segment_body,
        out_type=jax.ShapeDtypeStruct((N_REL, N_NODES, DIM), jnp.float32),
        mesh=mesh,
        scratch_types=[
            pltpu.VMEM((HALF_A,), jnp.int32),
            pltpu.VMEM((HALF_B,), jnp.int32),
            pltpu.VMEM((HALF_A,), jnp.int32),
            pltpu.VMEM((HALF_B,), jnp.int32),
            pltpu.VMEM((CHUNK,), jnp.int32),
            pltpu.VMEM((CHUNK,), jnp.int32),
            pltpu.VMEM((CHUNK,), jnp.int32),
            pltpu.VMEM((CHUNK,), jnp.int32),
            pltpu.VMEM((CHUNK,), jnp.float32),
            pltpu.VMEM((CHUNK,), jnp.float32),
            pltpu.VMEM((CHUNK, DIM), jnp.float32),
            pltpu.VMEM((CHUNK, DIM), jnp.float32),
            pltpu.VMEM((CHUNK, DIM), jnp.float32),
            pltpu.VMEM((CHUNK, DIM), jnp.float32),
            pltpu.VMEM_SHARED((N_NODES, DIM), jnp.float32),
            pltpu.SemaphoreType.DMA,
            pltpu.SemaphoreType.DMA,
            pltpu.SemaphoreType.DMA,
            pltpu.SemaphoreType.DMA,
            pltpu.SemaphoreType.DMA,
            pltpu.SemaphoreType.DMA,
            pltpu.SemaphoreType.DMA,
            pltpu.SemaphoreType.DMA,
            pltpu.SemaphoreType.DMA,
            pltpu.SemaphoreType.DMA,
            pltpu.SemaphoreType.DMA,
            pltpu.SemaphoreType.DMA,
            pltpu.SemaphoreType.DMA,
            pltpu.SemaphoreType.DMA,
        ],
    )(vertex, src, dst, ew)


def _tc_matmul_body(sup_ref, w_ref, wc_ref, b_ref, o_ref):
    c00 = wc_ref[0, 0]
    c01 = wc_ref[0, 1]
    c10 = wc_ref[1, 0]
    c11 = wc_ref[1, 1]
    a0 = w_ref[0:DIM, :]
    a1 = w_ref[DIM:2 * DIM, :]
    p0 = c00 * a0 + c01 * a1
    p1 = c10 * a0 + c11 * a1
    pp = jnp.concatenate([p0, p1], axis=0)
    rows_i = lax.broadcasted_iota(jnp.int32, (2 * DIM, 2 * DIM), 0)
    cols_i = lax.broadcasted_iota(jnp.int32, (2 * DIM, 2 * DIM), 1)
    g = ((rows_i % 2) * DIM + rows_i // 2 == cols_i).astype(jnp.float32)
    veff = jnp.dot(g, pp, preferred_element_type=jnp.float32)
    u0 = veff[0:DIM, :]
    u1 = veff[DIM:2 * DIM, :]
    acc = jnp.dot(sup_ref[0], u0, preferred_element_type=jnp.float32)
    acc = acc + jnp.dot(sup_ref[1], u1, preferred_element_type=jnp.float32)
    o_ref[...] = acc + b_ref[...]


@jax.jit
def _tc_matmul(supports, W, W_comp, B2d):
    blk = 2000
    return pl.pallas_call(
        _tc_matmul_body,
        grid=(N_NODES // blk,),
        in_specs=[
            pl.BlockSpec((N_REL, blk, DIM), lambda i: (0, i, 0)),
            pl.BlockSpec((2 * DIM, DIM), lambda i: (0, 0)),
            pl.BlockSpec(memory_space=pltpu.SMEM),
            pl.BlockSpec((1, DIM), lambda i: (0, 0)),
        ],
        out_specs=pl.BlockSpec((blk, DIM), lambda i: (i, 0)),
        out_shape=jax.ShapeDtypeStruct((N_NODES, DIM), jnp.float32),
    )(supports, W, W_comp, B2d)


def kernel(vertex, edge_index, edge_weight, W, W_comp, B):
    ei = edge_index.astype(jnp.int32)
    src = ei[:, 0, :].reshape(-1)
    dst = ei[:, 1, :].reshape(-1)
    supports = _sc_segment(vertex, src, dst, edge_weight.reshape(-1))
    return _tc_matmul(supports, W, W_comp, B.reshape(1, DIM))


# speedup vs baseline: 1.2397x; 1.1149x over previous
import jax
import jax.numpy as jnp
from jax import lax
from jax.experimental import pallas as pl
from jax.experimental.pallas import tpu as pltpu
from jax.experimental.pallas import tpu_sc as plsc

N_NODES = 10000
N_EDGES = 320000
N_REL = 2
DIM = 128
NUM_TILES = 16
EDGES_PER_TILE = N_EDGES // NUM_TILES
CHUNK = 80
N_CHUNKS = EDGES_PER_TILE // CHUNK
NBUF = 4
ROWS_PER_TILE = 624
TAIL_ROW = ROWS_PER_TILE * NUM_TILES
TAIL = N_NODES - TAIL_ROW


def _sc_segment_body(vertex, src, dst, ew, out,
                     s0, s1, d0, d1, d2, d3, w0, w1,
                     r0, r1, r2, r3, acc,
                     gA, gB, gC, gD, sA, sB, sC, sD, iA, iB):
    r = lax.axis_index("c")
    t = lax.axis_index("s")
    srcb = (s0, s1)
    dstb = (d0, d1, d2, d3)
    wb = (w0, w1)
    rows = (r0, r1, r2, r3)
    gsem = (gA, gB, gC, gD)
    ssem = (sA, sB, sC, sD)
    isem = (iA, iB)
    base = r * N_EDGES + t * EDGES_PER_TILE

    def zero_row(i, _):
        for f in range(DIM // 16):
            r0[i, pl.ds(f * 16, 16)] = jnp.zeros((16,), jnp.float32)
        return 0
    lax.fori_loop(0, CHUNK, zero_row, 0)
    row_base = t * ROWS_PER_TILE
    for k in range(7):
        pltpu.sync_copy(r0, acc.at[pl.ds(row_base + k * CHUNK, CHUNK)])
    pltpu.sync_copy(r0.at[pl.ds(0, 64)], acc.at[pl.ds(row_base + 560, 64)])

    @pl.when(t == 0)
    def _zero_tail():
        pltpu.sync_copy(r0.at[pl.ds(0, TAIL)], acc.at[pl.ds(TAIL_ROW, TAIL)])

    plsc.subcore_barrier()

    def issue_idx(c, i2, b4):
        off = base + c * CHUNK
        pltpu.async_copy(src.at[pl.ds(off, CHUNK)], srcb[i2], isem[i2])
        pltpu.async_copy(dst.at[pl.ds(off, CHUNK)], dstb[b4], isem[i2])
        pltpu.async_copy(ew.at[pl.ds(off, CHUNK)], wb[i2], isem[i2])

    def wait_idx(i2):
        pltpu.make_async_copy(src.at[pl.ds(0, CHUNK)], srcb[i2], isem[i2]).wait()
        pltpu.make_async_copy(dst.at[pl.ds(0, CHUNK)], dstb[0], isem[i2]).wait()
        pltpu.make_async_copy(ew.at[pl.ds(0, CHUNK)], wb[i2], isem[i2]).wait()

    def issue_gather(i2, b4):
        pltpu.async_copy(vertex.at[srcb[i2]], rows[b4], gsem[b4])

    def wait_gather(b4):
        pltpu.make_async_copy(vertex.at[pl.ds(0, CHUNK)], rows[b4],
                              gsem[b4]).wait()

    def issue_scatter(b4):
        pltpu.async_copy(rows[b4], acc.at[dstb[b4]], ssem[b4], add=True)

    def wait_scatter(b4):
        pltpu.make_async_copy(rows[b4], acc.at[pl.ds(0, CHUNK)],
                              ssem[b4]).wait()

    def scale(i2, b4, g_lo, g_hi):
        def grp(g2, _):
            w_vec = wb[i2][pl.ds(g2 * 16, 16)]
            for j in range(16):
                wj = w_vec[j]
                e = g2 * 16 + j
                for f in range(DIM // 16):
                    sl = pl.ds(f * 16, 16)
                    rows[b4][e, sl] = rows[b4][e, sl] * wj
            return 0
        lax.fori_loop(g_lo, g_hi, grp, 0)

    issue_idx(0, 0, 0)
    issue_idx(1, 1, 1)
    wait_idx(0)
    issue_gather(0, 0)

    def outer(g, _):
        for b in range(NBUF):
            c = NBUF * g + b
            i2 = b % 2
            inx = (b + 1) % 2

            @pl.when(c >= 2)
            def _wait_prev_scatter():
                wait_scatter((b + 2) % NBUF)

            @pl.when(c + 1 < N_CHUNKS)
            def _launch_next_gather():
                wait_idx(inx)
                issue_gather(inx, (b + 1) % NBUF)

            @pl.when(c < N_CHUNKS)
            def _scale_and_scatter():
                wait_gather(b)
                scale(i2, b, 0, CHUNK // 16)
                issue_scatter(b)

            @pl.when(c + 2 < N_CHUNKS)
            def _prefetch_idx():
                issue_idx(c + 2, i2, (b + 2) % NBUF)
        return 0

    lax.fori_loop(0, N_CHUNKS // NBUF + 1, outer, 0)
    plsc.subcore_barrier()

    for k in range(7):
        row0 = row_base + k * CHUNK
        pltpu.sync_copy(acc.at[pl.ds(row0, CHUNK)], r0)
        pltpu.sync_copy(r0, out.at[r, pl.ds(row0, CHUNK)])
    pltpu.sync_copy(acc.at[pl.ds(row_base + 560, 64)], r0.at[pl.ds(0, 64)])
    pltpu.sync_copy(r0.at[pl.ds(0, 64)], out.at[r, pl.ds(row_base + 560, 64)])

    @pl.when(t == 0)
    def _flush_tail():
        pltpu.sync_copy(acc.at[pl.ds(TAIL_ROW, TAIL)], r1.at[pl.ds(0, TAIL)])
        pltpu.sync_copy(r1.at[pl.ds(0, TAIL)], out.at[r, pl.ds(TAIL_ROW, TAIL)])


@jax.jit
def _sc_segment(vertex, src, dst, ew):
    mesh = plsc.VectorSubcoreMesh(core_axis_name="c", subcore_axis_name="s")
    return pl.kernel(
        _sc_segment_body,
        out_type=jax.ShapeDtypeStruct((N_REL, N_NODES, DIM), jnp.float32),
        mesh=mesh,
        scratch_types=[
            pltpu.VMEM((CHUNK,), jnp.int32),
            pltpu.VMEM((CHUNK,), jnp.int32),
            pltpu.VMEM((CHUNK,), jnp.int32),
            pltpu.VMEM((CHUNK,), jnp.int32),
            pltpu.VMEM((CHUNK,), jnp.int32),
            pltpu.VMEM((CHUNK,), jnp.int32),
            pltpu.VMEM((CHUNK,), jnp.float32),
            pltpu.VMEM((CHUNK,), jnp.float32),
            pltpu.VMEM((CHUNK, DIM), jnp.float32),
            pltpu.VMEM((CHUNK, DIM), jnp.float32),
            pltpu.VMEM((CHUNK, DIM), jnp.float32),
            pltpu.VMEM((CHUNK, DIM), jnp.float32),
            pltpu.VMEM_SHARED((N_NODES, DIM), jnp.float32),
            pltpu.SemaphoreType.DMA,
            pltpu.SemaphoreType.DMA,
            pltpu.SemaphoreType.DMA,
            pltpu.SemaphoreType.DMA,
            pltpu.SemaphoreType.DMA,
            pltpu.SemaphoreType.DMA,
            pltpu.SemaphoreType.DMA,
            pltpu.SemaphoreType.DMA,
            pltpu.SemaphoreType.DMA,
            pltpu.SemaphoreType.DMA,
        ],
    )(vertex, src, dst, ew)


def _tc_matmul_body(sup_ref, w_ref, wc_ref, b_ref, o_ref):
    c00 = wc_ref[0, 0]
    c01 = wc_ref[0, 1]
    c10 = wc_ref[1, 0]
    c11 = wc_ref[1, 1]
    a0 = w_ref[0:DIM, :]
    a1 = w_ref[DIM:2 * DIM, :]
    p0 = c00 * a0 + c01 * a1
    p1 = c10 * a0 + c11 * a1
    pp = jnp.concatenate([p0, p1], axis=0)
    rows_i = lax.broadcasted_iota(jnp.int32, (2 * DIM, 2 * DIM), 0)
    cols_i = lax.broadcasted_iota(jnp.int32, (2 * DIM, 2 * DIM), 1)
    g = ((rows_i % 2) * DIM + rows_i // 2 == cols_i).astype(jnp.float32)
    veff = jnp.dot(g, pp, preferred_element_type=jnp.float32)
    u0 = veff[0:DIM, :]
    u1 = veff[DIM:2 * DIM, :]
    acc = jnp.dot(sup_ref[0], u0, preferred_element_type=jnp.float32)
    acc = acc + jnp.dot(sup_ref[1], u1, preferred_element_type=jnp.float32)
    o_ref[...] = acc + b_ref[...]


@jax.jit
def _tc_matmul(supports, W, W_comp, B2d):
    blk = 2000
    return pl.pallas_call(
        _tc_matmul_body,
        grid=(N_NODES // blk,),
        in_specs=[
            pl.BlockSpec((N_REL, blk, DIM), lambda i: (0, i, 0)),
            pl.BlockSpec((2 * DIM, DIM), lambda i: (0, 0)),
            pl.BlockSpec(memory_space=pltpu.SMEM),
            pl.BlockSpec((1, DIM), lambda i: (0, 0)),
        ],
        out_specs=pl.BlockSpec((blk, DIM), lambda i: (i, 0)),
        out_shape=jax.ShapeDtypeStruct((N_NODES, DIM), jnp.float32),
    )(supports, W, W_comp, B2d)


def kernel(vertex, edge_index, edge_weight, W, W_comp, B):
    ei = edge_index.astype(jnp.int32)
    src = ei[:, 0, :].reshape(-1)
    dst = ei[:, 1, :].reshape(-1)
    supports = _sc_segment(vertex, src, dst, edge_weight.reshape(-1))
    return _tc_matmul(supports, W, W_comp, B.reshape(1, DIM))


# speedup vs baseline: 1.2439x; 1.0034x over previous
import jax
import jax.numpy as jnp
from jax import lax
from jax.experimental import pallas as pl
from jax.experimental.pallas import tpu as pltpu
from jax.experimental.pallas import tpu_sc as plsc

N_NODES = 10000
N_EDGES = 320000
N_REL = 2
DIM = 128
NUM_TILES = 16
EDGES_PER_TILE = N_EDGES // NUM_TILES
CHUNK = 80
N_CHUNKS = EDGES_PER_TILE // CHUNK
NBUF = 4
ROWS_PER_TILE = 624
TAIL_ROW = ROWS_PER_TILE * NUM_TILES
TAIL = N_NODES - TAIL_ROW


def _sc_segment_body(vertex, src, dst, ew, out,
                     s0, s1, d0, d1, d2, d3, w0, w1,
                     r0, r1, r2, r3, acc,
                     gA, gB, gC, gD, sA, sB, sC, sD, iA, iB):
    r = lax.axis_index("c")
    t = lax.axis_index("s")
    srcb = (s0, s1)
    dstb = (d0, d1, d2, d3)
    wb = (w0, w1)
    rows = (r0, r1, r2, r3)
    gsem = (gA, gB, gC, gD)
    ssem = (sA, sB, sC, sD)
    isem = (iA, iB)
    base = r * N_EDGES + t * EDGES_PER_TILE

    def zero_row(i, _):
        for f in range(DIM // 16):
            r0[i, pl.ds(f * 16, 16)] = jnp.zeros((16,), jnp.float32)
        return 0
    lax.fori_loop(0, CHUNK, zero_row, 0)
    row_base = t * ROWS_PER_TILE
    for k in range(7):
        pltpu.sync_copy(r0, acc.at[pl.ds(row_base + k * CHUNK, CHUNK)])
    pltpu.sync_copy(r0.at[pl.ds(0, 64)], acc.at[pl.ds(row_base + 560, 64)])

    @pl.when(t == 0)
    def _zero_tail():
        pltpu.sync_copy(r0.at[pl.ds(0, TAIL)], acc.at[pl.ds(TAIL_ROW, TAIL)])

    plsc.subcore_barrier()

    def issue_idx(c, i2, b4):
        off = base + c * CHUNK
        pltpu.async_copy(src.at[pl.ds(off, CHUNK)], srcb[i2], isem[i2])
        pltpu.async_copy(dst.at[pl.ds(off, CHUNK)], dstb[b4], isem[i2])
        pltpu.async_copy(ew.at[pl.ds(off, CHUNK)], wb[i2], isem[i2])

    def wait_idx(i2):
        pltpu.make_async_copy(src.at[pl.ds(0, CHUNK)], srcb[i2], isem[i2]).wait()
        pltpu.make_async_copy(dst.at[pl.ds(0, CHUNK)], dstb[0], isem[i2]).wait()
        pltpu.make_async_copy(ew.at[pl.ds(0, CHUNK)], wb[i2], isem[i2]).wait()

    def issue_gather(i2, b4):
        pltpu.async_copy(vertex.at[srcb[i2]], rows[b4], gsem[b4])

    def wait_gather(b4):
        pltpu.make_async_copy(vertex.at[pl.ds(0, CHUNK)], rows[b4],
                              gsem[b4]).wait()

    def issue_scatter(b4):
        pltpu.async_copy(rows[b4], acc.at[dstb[b4]], ssem[b4], add=True)

    def wait_scatter(b4):
        pltpu.make_async_copy(rows[b4], acc.at[pl.ds(0, CHUNK)],
                              ssem[b4]).wait()

    def scale(i2, b4, g_lo, g_hi):
        def grp(g2, _):
            w_vec = wb[i2][pl.ds(g2 * 16, 16)]
            for j in range(16):
                wj = lax.gather(
                    w_vec, jnp.full((16, 1), j, jnp.int32),
                    lax.GatherDimensionNumbers(
                        offset_dims=(), collapsed_slice_dims=(0,),
                        start_index_map=(0,)),
                    (1,), mode=lax.GatherScatterMode.PROMISE_IN_BOUNDS)
                e = g2 * 16 + j
                for f in range(DIM // 16):
                    sl = pl.ds(f * 16, 16)
                    rows[b4][e, sl] = rows[b4][e, sl] * wj
            return 0
        lax.fori_loop(g_lo, g_hi, grp, 0)

    issue_idx(0, 0, 0)
    issue_idx(1, 1, 1)
    wait_idx(0)
    issue_gather(0, 0)

    def outer(g, _):
        for b in range(NBUF):
            c = NBUF * g + b
            i2 = b % 2
            inx = (b + 1) % 2

            @pl.when(c >= 2)
            def _wait_prev_scatter():
                wait_scatter((b + 2) % NBUF)

            @pl.when(c + 1 < N_CHUNKS)
            def _launch_next_gather():
                wait_idx(inx)
                issue_gather(inx, (b + 1) % NBUF)

            @pl.when(c < N_CHUNKS)
            def _scale_and_scatter():
                wait_gather(b)
                scale(i2, b, 0, CHUNK // 16)
                issue_scatter(b)

            @pl.when(c + 2 < N_CHUNKS)
            def _prefetch_idx():
                issue_idx(c + 2, i2, (b + 2) % NBUF)
        return 0

    lax.fori_loop(0, N_CHUNKS // NBUF + 1, outer, 0)
    plsc.subcore_barrier()

    for k in range(7):
        row0 = row_base + k * CHUNK
        pltpu.sync_copy(acc.at[pl.ds(row0, CHUNK)], r0)
        pltpu.sync_copy(r0, out.at[r, pl.ds(row0, CHUNK)])
    pltpu.sync_copy(acc.at[pl.ds(row_base + 560, 64)], r0.at[pl.ds(0, 64)])
    pltpu.sync_copy(r0.at[pl.ds(0, 64)], out.at[r, pl.ds(row_base + 560, 64)])

    @pl.when(t == 0)
    def _flush_tail():
        pltpu.sync_copy(acc.at[pl.ds(TAIL_ROW, TAIL)], r1.at[pl.ds(0, TAIL)])
        pltpu.sync_copy(r1.at[pl.ds(0, TAIL)], out.at[r, pl.ds(TAIL_ROW, TAIL)])


@jax.jit
def _sc_segment(vertex, src, dst, ew):
    mesh = plsc.VectorSubcoreMesh(core_axis_name="c", subcore_axis_name="s")
    return pl.kernel(
        _sc_segment_body,
        out_type=jax.ShapeDtypeStruct((N_REL, N_NODES, DIM), jnp.float32),
        mesh=mesh,
        scratch_types=[
            pltpu.VMEM((CHUNK,), jnp.int32),
            pltpu.VMEM((CHUNK,), jnp.int32),
            pltpu.VMEM((CHUNK,), jnp.int32),
            pltpu.VMEM((CHUNK,), jnp.int32),
            pltpu.VMEM((CHUNK,), jnp.int32),
            pltpu.VMEM((CHUNK,), jnp.int32),
            pltpu.VMEM((CHUNK,), jnp.float32),
            pltpu.VMEM((CHUNK,), jnp.float32),
            pltpu.VMEM((CHUNK, DIM), jnp.float32),
            pltpu.VMEM((CHUNK, DIM), jnp.float32),
            pltpu.VMEM((CHUNK, DIM), jnp.float32),
            pltpu.VMEM((CHUNK, DIM), jnp.float32),
            pltpu.VMEM_SHARED((N_NODES, DIM), jnp.float32),
            pltpu.SemaphoreType.DMA,
            pltpu.SemaphoreType.DMA,
            pltpu.SemaphoreType.DMA,
            pltpu.SemaphoreType.DMA,
            pltpu.SemaphoreType.DMA,
            pltpu.SemaphoreType.DMA,
            pltpu.SemaphoreType.DMA,
            pltpu.SemaphoreType.DMA,
            pltpu.SemaphoreType.DMA,
            pltpu.SemaphoreType.DMA,
        ],
    )(vertex, src, dst, ew)


def _tc_matmul_body(sup_ref, w_ref, wc_ref, b_ref, o_ref):
    c00 = wc_ref[0, 0]
    c01 = wc_ref[0, 1]
    c10 = wc_ref[1, 0]
    c11 = wc_ref[1, 1]
    a0 = w_ref[0:DIM, :]
    a1 = w_ref[DIM:2 * DIM, :]
    p0 = c00 * a0 + c01 * a1
    p1 = c10 * a0 + c11 * a1
    pp = jnp.concatenate([p0, p1], axis=0)
    rows_i = lax.broadcasted_iota(jnp.int32, (2 * DIM, 2 * DIM), 0)
    cols_i = lax.broadcasted_iota(jnp.int32, (2 * DIM, 2 * DIM), 1)
    g = ((rows_i % 2) * DIM + rows_i // 2 == cols_i).astype(jnp.float32)
    veff = jnp.dot(g, pp, preferred_element_type=jnp.float32)
    u0 = veff[0:DIM, :]
    u1 = veff[DIM:2 * DIM, :]
    acc = jnp.dot(sup_ref[0], u0, preferred_element_type=jnp.float32)
    acc = acc + jnp.dot(sup_ref[1], u1, preferred_element_type=jnp.float32)
    o_ref[...] = acc + b_ref[...]


@jax.jit
def _tc_matmul(supports, W, W_comp, B2d):
    blk = 2000
    return pl.pallas_call(
        _tc_matmul_body,
        grid=(N_NODES // blk,),
        in_specs=[
            pl.BlockSpec((N_REL, blk, DIM), lambda i: (0, i, 0)),
            pl.BlockSpec((2 * DIM, DIM), lambda i: (0, 0)),
            pl.BlockSpec(memory_space=pltpu.SMEM),
            pl.BlockSpec((1, DIM), lambda i: (0, 0)),
        ],
        out_specs=pl.BlockSpec((blk, DIM), lambda i: (i, 0)),
        out_shape=jax.ShapeDtypeStruct((N_NODES, DIM), jnp.float32),
    )(supports, W, W_comp, B2d)


def kernel(vertex, edge_index, edge_weight, W, W_comp, B):
    ei = edge_index.astype(jnp.int32)
    src = ei[:, 0, :].reshape(-1)
    dst = ei[:, 1, :].reshape(-1)
    supports = _sc_segment(vertex, src, dst, edge_weight.reshape(-1))
    return _tc_matmul(supports, W, W_comp, B.reshape(1, DIM))
